# baseline (speedup 1.0000x reference)
import functools

import jax
import jax.numpy as jnp
from jax import lax
from jax.experimental import pallas as pl
from jax.experimental.pallas import tpu as pltpu
from jax.experimental.pallas import tpu_sc as plsc

VOCAB = 1_000_000
DIM = 64
WORLD_SIZE = 1
RANK = 0
BLOCK = (VOCAB + WORLD_SIZE - 1) // WORLD_SIZE
MIN_ID = RANK * BLOCK
MAX_ID = min(VOCAB, (RANK + 1) * BLOCK)
NULL_IDX = MAX_ID - MIN_ID

NC = 2
NS = 16
NW = NC * NS
LANES = 16

ROWS = NULL_IDX + 1
ROWS_PAD = 1_000_008
CB = 128
NCB = 7812
TAIL = ROWS - NCB * CB
WLIN = ROWS_PAD * DIM

TOKENS = 4096 * 50
BPW = TOKENS // NW
CHUNK = 128
NCHUNK = BPW // CHUNK
GROUP = 5
NGROUP = NCHUNK // GROUP

_CNT_LO, _SPLIT = 244, 28
_PAIRS = (_CNT_LO + 2) // 2


def _prep_body(wt_hbm, tail_hbm, out_hbm, in0, in1, ob0, ob1,
               sg0, sg1, so0, so1):
    wid = lax.axis_index("s") * NC + lax.axis_index("c")
    cnt = _CNT_LO + jnp.where(wid >= _SPLIT, 1, 0)
    base = _CNT_LO * wid + jnp.maximum(wid - _SPLIT, 0)

    @pl.when(wid == 0)
    def _():
        pltpu.sync_copy(tail_hbm, ob0.at[pl.ds(0, TAIL * DIM)])
        pltpu.sync_copy(ob0.at[pl.ds(0, TAIL * DIM)],
                        out_hbm.at[pl.ds(NCB * CB * DIM, TAIL * DIM)])

    iota = lax.iota(jnp.int32, LANES)
    row_mul = iota * DIM

    def fire_in(g, inb, sem):
        pltpu.async_copy(wt_hbm.at[:, pl.ds((base + g) * CB, CB)], inb, sem)

    def wait_in(inb, sem):
        pltpu.make_async_copy(wt_hbm.at[:, pl.ds(0, CB)], inb, sem).wait()

    def fire_out(g, ob, sem):
        pltpu.async_copy(ob, out_hbm.at[pl.ds((base + g) * CB * DIM, CB * DIM)],
                         sem)

    def wait_out(ob, sem):
        pltpu.make_async_copy(ob, out_hbm.at[pl.ds(0, CB * DIM)], sem).wait()

    def transpose(inb, ob):
        def tr(j, carry):
            col = jnp.full((LANES,), j, jnp.int32)
            for k in range(CB // LANES):
                vals = inb[j, pl.ds(k * LANES, LANES)]
                plsc.store_scatter(ob, [row_mul + (k * LANES * DIM + j)], vals)
            return carry

        lax.fori_loop(0, DIM, tr, None)

    fire_in(0, in0, sg0)
    fire_in(1, in1, sg1)

    def pair(p, carry):
        for g_off, inb, ob, sgi, soi in (
            (0, in0, ob0, sg0, so0),
            (1, in1, ob1, sg1, so1),
        ):
            g = 2 * p + g_off

            @pl.when(g < cnt)
            def _():
                wait_in(inb, sgi)

                @pl.when(g >= 2)
                def _():
                    wait_out(ob, soi)

                transpose(inb, ob)
                fire_out(g, ob, soi)

                @pl.when(g + 2 < cnt)
                def _():
                    fire_in(g + 2, inb, sgi)

        return carry

    lax.fori_loop(0, _PAIRS, pair, None)
    wait_out(ob0, so0)
    wait_out(ob1, so1)


def _gather_body(idx_hbm, w_hbm, out_hbm, idx_flat, idx_v, buf, sem):
    wid = lax.axis_index("s") * NC + lax.axis_index("c")
    pltpu.sync_copy(idx_hbm.at[pl.ds(wid * BPW, BPW)], idx_flat)

    def remap(t, carry):
        row = t // (CHUNK // LANES)
        col = (t % (CHUNK // LANES)) * LANES
        v = idx_flat[pl.ds(t * LANES, LANES)]
        oob = (v < MIN_ID) | (v >= MAX_ID)
        idx_v[row, pl.ds(col, LANES)] = jnp.where(oob, NULL_IDX, v - MIN_ID)
        return carry

    lax.fori_loop(0, BPW // LANES, remap, None)

    base = wid * BPW

    def group(g, carry):
        handles = []
        for b in range(GROUP):
            j = g * GROUP + b
            h = pltpu.async_copy(
                w_hbm.at[idx_v.at[j]], buf.at[pl.ds(b * CHUNK, CHUNK)], sem
            )
            handles.append(h)
        for h in handles:
            h.wait()
        pltpu.sync_copy(
            buf, out_hbm.at[pl.ds(base + g * (GROUP * CHUNK), GROUP * CHUNK)]
        )
        return carry

    lax.fori_loop(0, NGROUP, group, None)


@jax.jit
def kernel(input, weight):
    idx = input.astype(jnp.int32).reshape(TOKENS)
    wt = jnp.transpose(weight)
    wtail = lax.slice(weight, (NCB * CB, 0), (ROWS, DIM)).reshape(TAIL * DIM)
    mesh = plsc.VectorSubcoreMesh(
        core_axis_name="c", subcore_axis_name="s", num_cores=NC, num_subcores=NS
    )
    prep = functools.partial(
        pl.kernel,
        mesh=mesh,
        out_type=jax.ShapeDtypeStruct((WLIN,), jnp.float32),
        scratch_types=[
            pltpu.VMEM((DIM, CB), jnp.float32),
            pltpu.VMEM((DIM, CB), jnp.float32),
            pltpu.VMEM((CB * DIM,), jnp.float32),
            pltpu.VMEM((CB * DIM,), jnp.float32),
            pltpu.SemaphoreType.DMA,
            pltpu.SemaphoreType.DMA,
            pltpu.SemaphoreType.DMA,
            pltpu.SemaphoreType.DMA,
        ],
        compiler_params=pltpu.CompilerParams(
            use_tc_tiling_on_sc=True, needs_layout_passes=False
        ),
    )(_prep_body)
    wlin = prep(wt, wtail).reshape(ROWS_PAD, DIM)

    gather = functools.partial(
        pl.kernel,
        mesh=mesh,
        out_type=jax.ShapeDtypeStruct((TOKENS, DIM), jnp.float32),
        scratch_types=[
            pltpu.VMEM((BPW,), jnp.int32),
            pltpu.VMEM((NCHUNK, CHUNK), jnp.int32),
            pltpu.VMEM((GROUP * CHUNK, DIM), jnp.float32),
            pltpu.SemaphoreType.DMA,
        ],
        compiler_params=pltpu.CompilerParams(
            use_tc_tiling_on_sc=False, needs_layout_passes=False
        ),
    )(_gather_body)
    out = gather(idx, wlin)
    return out.reshape(input.shape[0], input.shape[1], DIM)


# speedup vs baseline: 1.9696x; 1.9696x over previous
import functools

import jax
import jax.numpy as jnp
from jax import lax
from jax.experimental import pallas as pl
from jax.experimental.pallas import tpu as pltpu
from jax.experimental.pallas import tpu_sc as plsc

VOCAB = 1_000_000
DIM = 64
WORLD_SIZE = 1
RANK = 0
BLOCK = (VOCAB + WORLD_SIZE - 1) // WORLD_SIZE
MIN_ID = RANK * BLOCK
MAX_ID = min(VOCAB, (RANK + 1) * BLOCK)
NULL_IDX = MAX_ID - MIN_ID

NC = 2
NS = 16
NW = NC * NS
LANES = 16

ROWS = NULL_IDX + 1
ROWS_PAD = 1_000_008
CB = 128
NCB = 7812
TAIL = ROWS - NCB * CB
WLIN = ROWS_PAD * DIM

TOKENS = 4096 * 50
BPW = TOKENS // NW
CHUNK = 128
NCHUNK = BPW // CHUNK
GROUP = 5
NGROUP = NCHUNK // GROUP

_CNT_LO, _SPLIT = 244, 28
_PAIRS = (_CNT_LO + 2) // 2


def _prep_body(wt_hbm, tail_hbm, out_hbm, in0, in1, ob0, ob1,
               sg0, sg1, so0, so1):
    wid = lax.axis_index("s") * NC + lax.axis_index("c")
    cnt = _CNT_LO + jnp.where(wid >= _SPLIT, 1, 0)
    base = _CNT_LO * wid + jnp.maximum(wid - _SPLIT, 0)

    @pl.when(wid == 0)
    def _():
        pltpu.sync_copy(tail_hbm, ob0.at[pl.ds(0, TAIL * DIM)])
        pltpu.sync_copy(ob0.at[pl.ds(0, TAIL * DIM)],
                        out_hbm.at[pl.ds(NCB * CB * DIM, TAIL * DIM)])

    iota = lax.iota(jnp.int32, LANES)
    rows_d = [(iota + d) % LANES for d in range(LANES)]
    sidx_d = [iota * DIM + rows_d[d] for d in range(LANES)]

    def fire_in(g, inb, sem):
        pltpu.async_copy(wt_hbm.at[:, pl.ds((base + g) * CB, CB)], inb, sem)

    def wait_in(inb, sem):
        pltpu.make_async_copy(wt_hbm.at[:, pl.ds(0, CB)], inb, sem).wait()

    def fire_out(g, ob, sem):
        pltpu.async_copy(ob, out_hbm.at[pl.ds((base + g) * CB * DIM, CB * DIM)],
                         sem)

    def wait_out(ob, sem):
        pltpu.make_async_copy(ob, out_hbm.at[pl.ds(0, CB * DIM)], sem).wait()

    def transpose(inb, ob):
        def tr(c_blk, carry):
            col_base = c_blk * LANES
            out_base = c_blk * (LANES * DIM)
            cols = iota + col_base
            for jb in range(DIM // LANES):
                for d in range(LANES):
                    vals = plsc.load_gather(inb, [rows_d[d] + jb * LANES, cols])
                    plsc.store_scatter(
                        ob, [sidx_d[d] + (out_base + jb * LANES)], vals
                    )
            return carry

        lax.fori_loop(0, CB // LANES, tr, None)

    fire_in(0, in0, sg0)
    fire_in(1, in1, sg1)

    def pair(p, carry):
        for g_off, inb, ob, sgi, soi in (
            (0, in0, ob0, sg0, so0),
            (1, in1, ob1, sg1, so1),
        ):
            g = 2 * p + g_off

            @pl.when(g < cnt)
            def _():
                wait_in(inb, sgi)

                @pl.when(g >= 2)
                def _():
                    wait_out(ob, soi)

                transpose(inb, ob)
                fire_out(g, ob, soi)

                @pl.when(g + 2 < cnt)
                def _():
                    fire_in(g + 2, inb, sgi)

        return carry

    lax.fori_loop(0, _PAIRS, pair, None)
    wait_out(ob0, so0)
    wait_out(ob1, so1)


def _gather_body(idx_hbm, w_hbm, out_hbm, idx_flat, idx_v, buf, sem):
    wid = lax.axis_index("s") * NC + lax.axis_index("c")
    pltpu.sync_copy(idx_hbm.at[pl.ds(wid * BPW, BPW)], idx_flat)

    def remap(t, carry):
        row = t // (CHUNK // LANES)
        col = (t % (CHUNK // LANES)) * LANES
        v = idx_flat[pl.ds(t * LANES, LANES)]
        oob = (v < MIN_ID) | (v >= MAX_ID)
        idx_v[row, pl.ds(col, LANES)] = jnp.where(oob, NULL_IDX, v - MIN_ID)
        return carry

    lax.fori_loop(0, BPW // LANES, remap, None)

    base = wid * BPW

    def group(g, carry):
        handles = []
        for b in range(GROUP):
            j = g * GROUP + b
            h = pltpu.async_copy(
                w_hbm.at[idx_v.at[j]], buf.at[pl.ds(b * CHUNK, CHUNK)], sem
            )
            handles.append(h)
        for h in handles:
            h.wait()
        pltpu.sync_copy(
            buf, out_hbm.at[pl.ds(base + g * (GROUP * CHUNK), GROUP * CHUNK)]
        )
        return carry

    lax.fori_loop(0, NGROUP, group, None)


@jax.jit
def kernel(input, weight):
    idx = input.astype(jnp.int32).reshape(TOKENS)
    wt = jnp.transpose(weight)
    wtail = lax.slice(weight, (NCB * CB, 0), (ROWS, DIM)).reshape(TAIL * DIM)
    mesh = plsc.VectorSubcoreMesh(
        core_axis_name="c", subcore_axis_name="s", num_cores=NC, num_subcores=NS
    )
    prep = functools.partial(
        pl.kernel,
        mesh=mesh,
        out_type=jax.ShapeDtypeStruct((WLIN,), jnp.float32),
        scratch_types=[
            pltpu.VMEM((DIM, CB), jnp.float32),
            pltpu.VMEM((DIM, CB), jnp.float32),
            pltpu.VMEM((CB * DIM,), jnp.float32),
            pltpu.VMEM((CB * DIM,), jnp.float32),
            pltpu.SemaphoreType.DMA,
            pltpu.SemaphoreType.DMA,
            pltpu.SemaphoreType.DMA,
            pltpu.SemaphoreType.DMA,
        ],
        compiler_params=pltpu.CompilerParams(
            use_tc_tiling_on_sc=True, needs_layout_passes=False
        ),
    )(_prep_body)
    wlin = prep(wt, wtail).reshape(ROWS_PAD, DIM)

    gather = functools.partial(
        pl.kernel,
        mesh=mesh,
        out_type=jax.ShapeDtypeStruct((TOKENS, DIM), jnp.float32),
        scratch_types=[
            pltpu.VMEM((BPW,), jnp.int32),
            pltpu.VMEM((NCHUNK, CHUNK), jnp.int32),
            pltpu.VMEM((GROUP * CHUNK, DIM), jnp.float32),
            pltpu.SemaphoreType.DMA,
        ],
        compiler_params=pltpu.CompilerParams(
            use_tc_tiling_on_sc=False, needs_layout_passes=False
        ),
    )(_gather_body)
    out = gather(idx, wlin)
    return out.reshape(input.shape[0], input.shape[1], DIM)
